# Initial kernel scaffold; baseline (speedup 1.0000x reference)
import jax
import jax.numpy as jnp
from jax.experimental import pallas as pl


def kernel(x, edge_index, W1, b1, W2, b2):
    raise NotImplementedError("write your pallas kernel here")


# speedup vs baseline: 13.9105x; 13.9105x over previous
import functools

import jax
import jax.numpy as jnp
from jax import lax
from jax.experimental import pallas as pl
from jax.experimental.pallas import tpu as pltpu
from jax.experimental.pallas import tpu_sc as plsc

NC = 2
NS = 16
NW = NC * NS
LB = 128


@functools.lru_cache(maxsize=None)
def _build(N, D, E):
    DP = 128
    NP = ((N // 128) + 1) * 128
    DUMMY = N
    RPS = NP // NS
    EW = ((E + NW * LB - 1) // (NW * LB)) * LB
    KE = EW // LB
    EP = EW * NW

    mesh = plsc.VectorSubcoreMesh(
        core_axis_name="c", subcore_axis_name="s", num_cores=NC, num_subcores=NS
    )

    def _deg_body(dst_hbm, z8_hbm, ones_hbm, out_hbm, dst_v, ones_v, acc):
        c = lax.axis_index("c")
        s = lax.axis_index("s")
        wid = s * NC + c
        pltpu.sync_copy(z8_hbm.at[pl.ds(s * RPS, RPS)], acc.at[pl.ds(s * RPS, RPS)])
        pltpu.sync_copy(ones_hbm, ones_v)
        plsc.subcore_barrier()

        @pl.loop(0, KE)
        def _(j):
            pltpu.sync_copy(dst_hbm.at[wid, j], dst_v.at[0])
            pltpu.sync_copy(ones_v, acc.at[dst_v.at[0]], add=True)

        plsc.subcore_barrier()
        pltpu.sync_copy(acc.at[pl.ds(s * RPS, RPS)], out_hbm.at[c, pl.ds(s * RPS, RPS)])

    deg_call = pl.kernel(
        _deg_body,
        out_type=jax.ShapeDtypeStruct((NC, NP, DP), jnp.float32),
        mesh=mesh,
        scratch_types=[
            pltpu.VMEM((1, LB), jnp.int32),
            pltpu.VMEM((LB, DP), jnp.float32),
            pltpu.VMEM_SHARED((NP, DP), jnp.float32),
        ],
    )

    def _prop_body(hp_hbm, src_hbm, dst_hbm, z_hbm, out_hbm, src_v, dst_v, rows_v, sem, acc):
        c = lax.axis_index("c")
        s = lax.axis_index("s")
        wid = s * NC + c
        pltpu.sync_copy(z_hbm.at[pl.ds(s * RPS, RPS)], acc.at[pl.ds(s * RPS, RPS)])
        plsc.subcore_barrier()

        @pl.loop(0, KE)
        def _(j):
            pltpu.sync_copy(src_hbm.at[wid, j], src_v.at[0])
            pltpu.sync_copy(dst_hbm.at[wid, j], dst_v.at[0])
            pltpu.async_copy(hp_hbm.at[src_v.at[0]], rows_v, sem).wait()
            pltpu.sync_copy(rows_v, acc.at[dst_v.at[0]], add=True)

        plsc.subcore_barrier()
        pltpu.sync_copy(acc.at[pl.ds(s * RPS, RPS)], out_hbm.at[c, pl.ds(s * RPS, RPS)])

    prop_call = pl.kernel(
        _prop_body,
        out_type=jax.ShapeDtypeStruct((NC, NP, DP), jnp.float32),
        mesh=mesh,
        scratch_types=[
            pltpu.VMEM((1, LB), jnp.int32),
            pltpu.VMEM((1, LB), jnp.int32),
            pltpu.VMEM((LB, DP), jnp.float32),
            pltpu.SemaphoreType.DMA,
            pltpu.VMEM_SHARED((NP, DP), jnp.float32),
        ],
    )

    def _dinv(degp_ref):
        deg = degp_ref[0][:, 0:1] + degp_ref[1][:, 0:1] + 1.0
        return lax.rsqrt(deg)

    def _tc1_body(xp_ref, w_ref, degp_ref, out_ref):
        h = jnp.dot(xp_ref[...], w_ref[...], preferred_element_type=jnp.float32,
                    precision=lax.Precision.HIGHEST)
        out_ref[...] = h * _dinv(degp_ref)

    tc1_call = pl.pallas_call(
        _tc1_body,
        out_shape=jax.ShapeDtypeStruct((NP, DP), jnp.float32),
    )

    def _tc2_body(acc_ref, hp_ref, degp_ref, b_ref, w_ref, out_ref):
        dinv = _dinv(degp_ref)
        srow = acc_ref[0] + acc_ref[1] + hp_ref[...]
        pre = srow * dinv + b_ref[...]
        h2 = jnp.maximum(pre, 0.0)
        out_ref[...] = jnp.dot(h2, w_ref[...], preferred_element_type=jnp.float32,
                               precision=lax.Precision.HIGHEST) * dinv

    tc2_call = pl.pallas_call(
        _tc2_body,
        out_shape=jax.ShapeDtypeStruct((NP, DP), jnp.float32),
    )

    def _tc3_body(acc_ref, hp_ref, degp_ref, b_ref, out_ref):
        dinv = _dinv(degp_ref)
        srow = acc_ref[0] + acc_ref[1] + hp_ref[...]
        o = srow * dinv + b_ref[...]
        col = lax.broadcasted_iota(jnp.int32, (NP, DP), 1)
        om = jnp.where(col < D, o, -jnp.inf)
        m = jnp.max(om, axis=1, keepdims=True)
        lse = jnp.log(jnp.sum(jnp.exp(om - m), axis=1, keepdims=True)) + m
        out_ref[...] = o - lse

    tc3_call = pl.pallas_call(
        _tc3_body,
        out_shape=jax.ShapeDtypeStruct((NP, DP), jnp.float32),
    )

    @jax.jit
    def run(x, edge_index, W1, b1, W2, b2):
        f32 = jnp.float32
        xp = jnp.pad(x.astype(f32), ((0, NP - N), (0, DP - D)))
        W1p = jnp.pad(W1.astype(f32), ((0, DP - D), (0, DP - D)))
        W2p = jnp.pad(W2.astype(f32), ((0, DP - D), (0, DP - D)))
        b1p = jnp.pad(b1.astype(f32), (0, DP - D)).reshape(1, DP)
        b2p = jnp.pad(b2.astype(f32), (0, DP - D)).reshape(1, DP)
        ei = edge_index.astype(jnp.int32)
        src3 = jnp.pad(ei[0], (0, EP - E), constant_values=DUMMY).reshape(NW, KE, LB)
        dst3 = jnp.pad(ei[1], (0, EP - E), constant_values=DUMMY).reshape(NW, KE, LB)
        z128 = jnp.zeros((NP, DP), f32)
        ones128 = jnp.ones((LB, DP), f32)

        degp = deg_call(dst3, z128, ones128)
        h1p = tc1_call(xp, W1p, degp)
        acc1 = prop_call(h1p, src3, dst3, z128)
        h2p = tc2_call(acc1, h1p, degp, b1p, W2p)
        acc2 = prop_call(h2p, src3, dst3, z128)
        outp = tc3_call(acc2, h2p, degp, b2p)
        return outp[:N, :D]

    return run


def kernel(x, edge_index, W1, b1, W2, b2):
    run = _build(x.shape[0], x.shape[1], edge_index.shape[1])
    return run(x, edge_index, W1, b1, W2, b2)
